# baseline (speedup 1.0000x reference)
import functools

import jax
import jax.numpy as jnp
from jax import lax
from jax.experimental import pallas as pl
from jax.experimental.pallas import tpu as pltpu
from jax.experimental.pallas import tpu_sc as plsc

_BK = 1792
_TOPK = 4
_BIGI = 2**30


def _dist_topk_body(q_ref, m_ref, w_ref, idx_ref, vals_s, idx_s, qsq_s,
                    d2a_s, d2b_s, *, nk, bk, n_valid):
    k = pl.program_id(0)
    par = lax.rem(k, 2)

    @pl.when(k == 0)
    def _init():
        vals_s[...] = jnp.full(vals_s.shape, jnp.inf, jnp.float32)
        idx_s[...] = jnp.zeros(idx_s.shape, jnp.int32)
        q0 = q_ref[...]
        ones = jnp.ones((8, q0.shape[1]), jnp.float32)
        qsq_s[...] = lax.dot_general(ones, q0 * q0, (((1,), (1,)), ((), ())),
                                     preferred_element_type=jnp.float32)
        d2b_s[...] = jnp.full(d2b_s.shape, jnp.inf, jnp.float32)

    q = q_ref[...]
    Q = q.shape[0]
    nch = 4
    ch = bk // nch

    def _phase(wr_ref, rd_ref):
        def emit_mm_chunk(c):
            mc = m_ref[ch * c:ch * (c + 1), :]
            msq_c = jnp.sum(mc * mc, axis=1, keepdims=True)
            mm_c = lax.dot_general(mc, q, (((1,), (1,)), ((), ())),
                                   preferred_element_type=jnp.float32,
                                   precision=lax.Precision.DEFAULT)
            d2_c = (qsq_s[0:1, :] + msq_c) - 2.0 * mm_c
            rows_c = lax.broadcasted_iota(jnp.int32, d2_c.shape, 0)
            d2_c = jnp.where(rows_c + (k * bk + c * ch) < n_valid,
                             d2_c, jnp.inf)
            wr_ref[ch * c:ch * (c + 1), :] = d2_c

        base_prev = (k - 1) * bk
        ngr = Q // 128
        emit_mm_chunk(0)
        emitted = 1
        for g in range(ngr):
            lo, hi = 128 * g, 128 * (g + 1)

            def slab_update(i, slab, carry):
                a1, a2, a3, a4, i1, i2, i3, i4 = carry
                v = rd_ref[pl.ds(8 * i + 8 * slab, 8), lo:hi]
                rid = (lax.broadcasted_iota(jnp.int32, (8, 128), 0)
                       + (base_prev + 8 * i + 8 * slab))
                c1 = v < a1
                c2 = v < a2
                c3 = v < a3
                c4 = v < a4
                a4n = jnp.where(c3, a3, jnp.where(c4, v, a4))
                i4n = jnp.where(c3, i3, jnp.where(c4, rid, i4))
                a3n = jnp.where(c2, a2, jnp.where(c3, v, a3))
                i3n = jnp.where(c2, i2, jnp.where(c3, rid, i3))
                a2n = jnp.where(c1, a1, jnp.where(c2, v, a2))
                i2n = jnp.where(c1, i1, jnp.where(c2, rid, i2))
                a1n = jnp.where(c1, v, a1)
                i1n = jnp.where(c1, rid, i1)
                return (a1n, a2n, a3n, a4n, i1n, i2n, i3n, i4n)

            def body(j, carry):
                i = j * 2
                carry = slab_update(i, 0, carry)
                carry = slab_update(i, 1, carry)
                return carry

            inf8 = jnp.full((8, 128), jnp.inf, jnp.float32)
            z8 = jnp.zeros((8, 128), jnp.int32)
            a1, a2, a3, a4, i1, i2, i3, i4 = lax.fori_loop(
                0, bk // 16, body, (inf8, inf8, inf8, inf8, z8, z8, z8, z8))
            vals_s[8:16, lo:hi] = a1
            vals_s[16:24, lo:hi] = a2
            vals_s[24:32, lo:hi] = a3
            vals_s[32:40, lo:hi] = a4
            idx_s[8:16, lo:hi] = i1
            idx_s[16:24, lo:hi] = i2
            idx_s[24:32, lo:hi] = i3
            idx_s[32:40, lo:hi] = i4
            while emitted < nch and emitted <= ((g + 1) * nch) // ngr:
                emit_mm_chunk(emitted)
                emitted += 1
        while emitted < nch:
            emit_mm_chunk(emitted)
            emitted += 1

    @pl.when(par == 0)
    def _even():
        _phase(d2a_s, d2b_s)

    @pl.when(par == 1)
    def _odd():
        _phase(d2b_s, d2a_s)

    wv = vals_s[...]
    wi = idx_s[...]
    new_v, new_i = [], []
    for t in range(_TOPK):
        mv = jnp.min(wv, axis=0, keepdims=True)
        veq = wv == mv
        mi = jnp.min(jnp.where(veq, wi, _BIGI), axis=0, keepdims=True)
        new_v.append(mv)
        new_i.append(mi)
        if t < _TOPK - 1:
            wv = jnp.where(veq & (wi == mi), jnp.inf, wv)
    vals_s[0:4, :] = jnp.concatenate(new_v, axis=0)
    idx_s[0:4, :] = jnp.concatenate(new_i, axis=0)

    @pl.when(k == nk)
    def _finish():
        v4 = vals_s[0:4, :]
        dist = jnp.sqrt(jnp.maximum(v4, 1e-12))
        e = jnp.exp(dist[0:1, :] - dist)
        w = e / jnp.sum(e, axis=0, keepdims=True)
        w_ref[0:4, :] = w
        w_ref[4:8, :] = jnp.zeros((4, w.shape[1]), jnp.float32)
        idx_ref[0:4, :] = idx_s[0:4, :]
        idx_ref[4:8, :] = jnp.zeros((4, w.shape[1]), jnp.int32)


def _dist_topk(q, m):
    Q, D = q.shape
    n_valid = m.shape[0]
    nk = (n_valid + _BK - 1) // _BK
    body = functools.partial(_dist_topk_body, nk=nk, bk=_BK, n_valid=n_valid)
    return pl.pallas_call(
        body,
        grid=(nk + 1,),
        in_specs=[
            pl.BlockSpec((Q, D), lambda k: (0, 0)),
            pl.BlockSpec((_BK, D), lambda k: (jnp.minimum(k, nk - 1), 0)),
        ],
        out_specs=[
            pl.BlockSpec((8, Q), lambda k: (0, 0)),
            pl.BlockSpec((8, Q), lambda k: (0, 0)),
        ],
        out_shape=[
            jax.ShapeDtypeStruct((8, Q), jnp.float32),
            jax.ShapeDtypeStruct((8, Q), jnp.int32),
        ],
        scratch_shapes=[
            pltpu.VMEM((40, Q), jnp.float32),
            pltpu.VMEM((40, Q), jnp.int32),
            pltpu.VMEM((8, Q), jnp.float32),
            pltpu.VMEM((_BK, Q), jnp.float32),
            pltpu.VMEM((_BK, Q), jnp.float32),
        ],
        compiler_params=pltpu.CompilerParams(
            dimension_semantics=("arbitrary",)),
    )(q, m)


def _sc_gather(table, idx_flat):
    B = idx_flat.shape[0]
    D = table.shape[1]
    info = plsc.get_sparse_core_info()
    nw = info.num_cores * info.num_subcores
    b_per_w = B // nw
    ch = 32
    nch = b_per_w // ch
    mesh = plsc.VectorSubcoreMesh(core_axis_name="c", subcore_axis_name="s")

    @functools.partial(
        pl.kernel, mesh=mesh,
        out_type=jax.ShapeDtypeStruct((B, D), jnp.float32),
        scratch_types=[
            pltpu.VMEM((ch,), jnp.int32),
            pltpu.VMEM((ch, D), jnp.float32),
            pltpu.SemaphoreType.DMA,
        ],
    )
    def k(table_hbm, idx_hbm, out_hbm, idx_v, rows_v, sem):
        wid = lax.axis_index("s") * info.num_cores + lax.axis_index("c")
        for c in range(nch):
            base = wid * b_per_w + c * ch
            pltpu.sync_copy(idx_hbm.at[pl.ds(base, ch)], idx_v)
            pltpu.async_copy(table_hbm.at[idx_v], rows_v, sem).wait()
            pltpu.sync_copy(rows_v, out_hbm.at[pl.ds(base, ch)])

    return k(table, idx_flat)


def _wsum_body(g_ref, w_ref, o_ref):
    w = w_ref[...]
    acc = w[:, 0:1] * g_ref[0]
    for j in range(1, _TOPK):
        acc = acc + w[:, j:j + 1] * g_ref[j]
    o_ref[...] = acc


def _wsum(g4, w_q):
    _, Q, D = g4.shape
    BQ = 256
    return pl.pallas_call(
        _wsum_body,
        grid=(Q // BQ,),
        in_specs=[
            pl.BlockSpec((_TOPK, BQ, D), lambda i: (0, i, 0)),
            pl.BlockSpec((BQ, 8), lambda i: (i, 0)),
        ],
        out_specs=pl.BlockSpec((BQ, D), lambda i: (i, 0)),
        out_shape=jax.ShapeDtypeStruct((Q, D), jnp.float32),
    )(g4, w_q)


def kernel(query_seq, matching_set, synth_set, topk):
    Q, D = query_seq.shape
    w8, idx8 = _dist_topk(query_seq, matching_set)
    idx_flat = idx8[0:_TOPK, :].reshape(_TOPK * Q)
    g = _sc_gather(synth_set, idx_flat)
    g4 = g.reshape(_TOPK, Q, D)
    out = _wsum(g4, w8.T)
    return out


# speedup vs baseline: 1.1466x; 1.1466x over previous
import functools

import jax
import jax.numpy as jnp
from jax import lax
from jax.experimental import pallas as pl
from jax.experimental.pallas import tpu as pltpu
from jax.experimental.pallas import tpu_sc as plsc

_BK = 1792
_TOPK = 4
_BIGI = 2**30


def _dist_topk_body(q_ref, m_ref, w_ref, idx_ref, vals_s, idx_s, qsq_s,
                    d2a_s, d2b_s, *, nk, bk, n_valid):
    k = pl.program_id(0)
    par = lax.rem(k, 2)

    @pl.when(k == 0)
    def _init():
        vals_s[...] = jnp.full(vals_s.shape, jnp.inf, jnp.float32)
        idx_s[...] = jnp.zeros(idx_s.shape, jnp.int32)
        q0 = q_ref[...]
        ones = jnp.ones((8, q0.shape[1]), jnp.float32)
        qsq_s[...] = lax.dot_general(ones, q0 * q0, (((1,), (1,)), ((), ())),
                                     preferred_element_type=jnp.float32)
        d2b_s[...] = jnp.full(d2b_s.shape, jnp.inf, jnp.float32)

    q = q_ref[...]
    Q = q.shape[0]
    nch = 4
    ch = bk // nch

    def _phase(wr_ref, rd_ref):
        def emit_mm_chunk(c):
            mc = m_ref[ch * c:ch * (c + 1), :]
            msq_c = jnp.sum(mc * mc, axis=1, keepdims=True)
            mm_c = lax.dot_general(mc, q, (((1,), (1,)), ((), ())),
                                   preferred_element_type=jnp.float32,
                                   precision=lax.Precision.DEFAULT)
            d2_c = (qsq_s[0:1, :] + msq_c) - 2.0 * mm_c
            rows_c = lax.broadcasted_iota(jnp.int32, d2_c.shape, 0)
            d2_c = jnp.where(rows_c + (k * bk + c * ch) < n_valid,
                             d2_c, jnp.inf)
            wr_ref[ch * c:ch * (c + 1), :] = d2_c

        base_prev = (k - 1) * bk
        ngr = Q // 128
        emit_mm_chunk(0)
        emitted = 1
        for g in range(ngr):
            lo, hi = 128 * g, 128 * (g + 1)

            def slab_update(i, slab, carry):
                a1, a2, a3, a4, i1, i2, i3, i4 = carry
                v = rd_ref[pl.ds(8 * i + 8 * slab, 8), lo:hi]
                rid = (lax.broadcasted_iota(jnp.int32, (8, 128), 0)
                       + (base_prev + 8 * i + 8 * slab))
                c1 = v < a1
                c2 = v < a2
                c3 = v < a3
                c4 = v < a4
                a4n = jnp.where(c3, a3, jnp.where(c4, v, a4))
                i4n = jnp.where(c3, i3, jnp.where(c4, rid, i4))
                a3n = jnp.where(c2, a2, jnp.where(c3, v, a3))
                i3n = jnp.where(c2, i2, jnp.where(c3, rid, i3))
                a2n = jnp.where(c1, a1, jnp.where(c2, v, a2))
                i2n = jnp.where(c1, i1, jnp.where(c2, rid, i2))
                a1n = jnp.where(c1, v, a1)
                i1n = jnp.where(c1, rid, i1)
                return (a1n, a2n, a3n, a4n, i1n, i2n, i3n, i4n)

            def body(j, carry):
                i = j * 4
                for s in range(4):
                    carry = slab_update(i, s, carry)
                return carry

            inf8 = jnp.full((8, 128), jnp.inf, jnp.float32)
            z8 = jnp.zeros((8, 128), jnp.int32)
            a1, a2, a3, a4, i1, i2, i3, i4 = lax.fori_loop(
                0, bk // 32, body, (inf8, inf8, inf8, inf8, z8, z8, z8, z8))
            vals_s[8:16, lo:hi] = a1
            vals_s[16:24, lo:hi] = a2
            vals_s[24:32, lo:hi] = a3
            vals_s[32:40, lo:hi] = a4
            idx_s[8:16, lo:hi] = i1
            idx_s[16:24, lo:hi] = i2
            idx_s[24:32, lo:hi] = i3
            idx_s[32:40, lo:hi] = i4
            while emitted < nch and emitted <= ((g + 1) * nch) // ngr:
                emit_mm_chunk(emitted)
                emitted += 1
        while emitted < nch:
            emit_mm_chunk(emitted)
            emitted += 1

    @pl.when(par == 0)
    def _even():
        _phase(d2a_s, d2b_s)

    @pl.when(par == 1)
    def _odd():
        _phase(d2b_s, d2a_s)

    wv = vals_s[...]
    wi = idx_s[...]
    new_v, new_i = [], []
    for t in range(_TOPK):
        mv = jnp.min(wv, axis=0, keepdims=True)
        veq = wv == mv
        mi = jnp.min(jnp.where(veq, wi, _BIGI), axis=0, keepdims=True)
        new_v.append(mv)
        new_i.append(mi)
        if t < _TOPK - 1:
            wv = jnp.where(veq & (wi == mi), jnp.inf, wv)
    vals_s[0:4, :] = jnp.concatenate(new_v, axis=0)
    idx_s[0:4, :] = jnp.concatenate(new_i, axis=0)

    @pl.when(k == nk)
    def _finish():
        v4 = vals_s[0:4, :]
        dist = jnp.sqrt(jnp.maximum(v4, 1e-12))
        e = jnp.exp(dist[0:1, :] - dist)
        w = e / jnp.sum(e, axis=0, keepdims=True)
        w_ref[0:4, :] = w
        w_ref[4:8, :] = jnp.zeros((4, w.shape[1]), jnp.float32)
        idx_ref[0:4, :] = idx_s[0:4, :]
        idx_ref[4:8, :] = jnp.zeros((4, w.shape[1]), jnp.int32)


def _dist_topk(q, m):
    Q, D = q.shape
    n_valid = m.shape[0]
    nk = (n_valid + _BK - 1) // _BK
    body = functools.partial(_dist_topk_body, nk=nk, bk=_BK, n_valid=n_valid)
    return pl.pallas_call(
        body,
        grid=(nk + 1,),
        in_specs=[
            pl.BlockSpec((Q, D), lambda k: (0, 0)),
            pl.BlockSpec((_BK, D), lambda k: (jnp.minimum(k, nk - 1), 0)),
        ],
        out_specs=[
            pl.BlockSpec((8, Q), lambda k: (0, 0)),
            pl.BlockSpec((8, Q), lambda k: (0, 0)),
        ],
        out_shape=[
            jax.ShapeDtypeStruct((8, Q), jnp.float32),
            jax.ShapeDtypeStruct((8, Q), jnp.int32),
        ],
        scratch_shapes=[
            pltpu.VMEM((40, Q), jnp.float32),
            pltpu.VMEM((40, Q), jnp.int32),
            pltpu.VMEM((8, Q), jnp.float32),
            pltpu.VMEM((_BK, Q), jnp.float32),
            pltpu.VMEM((_BK, Q), jnp.float32),
        ],
        compiler_params=pltpu.CompilerParams(
            dimension_semantics=("arbitrary",)),
    )(q, m)


def _sc_gather(table, idx_flat):
    B = idx_flat.shape[0]
    D = table.shape[1]
    info = plsc.get_sparse_core_info()
    nw = info.num_cores * info.num_subcores
    b_per_w = B // nw
    ch = 32
    nch = b_per_w // ch
    mesh = plsc.VectorSubcoreMesh(core_axis_name="c", subcore_axis_name="s")

    @functools.partial(
        pl.kernel, mesh=mesh,
        out_type=jax.ShapeDtypeStruct((B, D), jnp.float32),
        scratch_types=[
            pltpu.VMEM((ch,), jnp.int32),
            pltpu.VMEM((ch, D), jnp.float32),
            pltpu.SemaphoreType.DMA,
        ],
    )
    def k(table_hbm, idx_hbm, out_hbm, idx_v, rows_v, sem):
        wid = lax.axis_index("s") * info.num_cores + lax.axis_index("c")
        for c in range(nch):
            base = wid * b_per_w + c * ch
            pltpu.sync_copy(idx_hbm.at[pl.ds(base, ch)], idx_v)
            pltpu.async_copy(table_hbm.at[idx_v], rows_v, sem).wait()
            pltpu.sync_copy(rows_v, out_hbm.at[pl.ds(base, ch)])

    return k(table, idx_flat)


def _wsum_body(g_ref, w_ref, o_ref):
    w = w_ref[...]
    acc = w[:, 0:1] * g_ref[0]
    for j in range(1, _TOPK):
        acc = acc + w[:, j:j + 1] * g_ref[j]
    o_ref[...] = acc


def _wsum(g4, w_q):
    _, Q, D = g4.shape
    BQ = 256
    return pl.pallas_call(
        _wsum_body,
        grid=(Q // BQ,),
        in_specs=[
            pl.BlockSpec((_TOPK, BQ, D), lambda i: (0, i, 0)),
            pl.BlockSpec((BQ, 8), lambda i: (i, 0)),
        ],
        out_specs=pl.BlockSpec((BQ, D), lambda i: (i, 0)),
        out_shape=jax.ShapeDtypeStruct((Q, D), jnp.float32),
    )(g4, w_q)


def kernel(query_seq, matching_set, synth_set, topk):
    Q, D = query_seq.shape
    w8, idx8 = _dist_topk(query_seq, matching_set)
    idx_flat = idx8[0:_TOPK, :].reshape(_TOPK * Q)
    g = _sc_gather(synth_set, idx_flat)
    g4 = g.reshape(_TOPK, Q, D)
    out = _wsum(g4, w8.T)
    return out


# speedup vs baseline: 1.2055x; 1.0514x over previous
import functools

import jax
import jax.numpy as jnp
from jax import lax
from jax.experimental import pallas as pl
from jax.experimental.pallas import tpu as pltpu
from jax.experimental.pallas import tpu_sc as plsc

_BK = 1792
_TOPK = 4
_BIGI = 2**30


def _dist_topk_body(q_ref, m_ref, w_ref, idx_ref, vals_s, idx_s, qsq_s,
                    d2a_s, d2b_s, *, nk, bk, n_valid):
    k = pl.program_id(0)
    par = lax.rem(k, 2)

    @pl.when(k == 0)
    def _init():
        vals_s[...] = jnp.full(vals_s.shape, jnp.inf, jnp.float32)
        idx_s[...] = jnp.zeros(idx_s.shape, jnp.int32)
        q0 = q_ref[...]
        ones = jnp.ones((8, q0.shape[1]), jnp.float32)
        qsq_s[...] = lax.dot_general(ones, q0 * q0, (((1,), (1,)), ((), ())),
                                     preferred_element_type=jnp.float32)
        d2b_s[...] = jnp.full(d2b_s.shape, jnp.inf, jnp.float32)

    q = q_ref[...]
    Q = q.shape[0]
    nch = 4
    ch = bk // nch

    def _phase(wr_ref, rd_ref):
        def emit_mm_chunk(c):
            mc = m_ref[ch * c:ch * (c + 1), :]
            msq_c = jnp.sum(mc * mc, axis=1, keepdims=True)
            mm_c = lax.dot_general(mc, q, (((1,), (1,)), ((), ())),
                                   preferred_element_type=jnp.float32,
                                   precision=lax.Precision.DEFAULT)
            d2_c = (qsq_s[0:1, :] + msq_c) - 2.0 * mm_c
            rows_c = lax.broadcasted_iota(jnp.int32, d2_c.shape, 0)
            d2_c = jnp.where(rows_c + (k * bk + c * ch) < n_valid,
                             d2_c, jnp.inf)
            wr_ref[ch * c:ch * (c + 1), :] = d2_c

        work = rd_ref[...]
        rows = lax.broadcasted_iota(jnp.int32, work.shape, 0)
        base_prev = (k - 1) * bk
        emit_mm_chunk(0)
        for t in range(_TOPK):
            mv = jnp.min(work, axis=0, keepdims=True)
            r = jnp.min(jnp.where(work == mv, rows, _BIGI),
                        axis=0, keepdims=True)
            vals_s[4 + t:5 + t, :] = mv
            idx_s[4 + t:5 + t, :] = r + base_prev
            if t < _TOPK - 1:
                work = jnp.where(rows == r, jnp.inf, work)
                emit_mm_chunk(t + 1)

    @pl.when(par == 0)
    def _even():
        _phase(d2a_s, d2b_s)

    @pl.when(par == 1)
    def _odd():
        _phase(d2b_s, d2a_s)

    wv = vals_s[...]
    wi = idx_s[...]
    rows8 = lax.broadcasted_iota(jnp.int32, wv.shape, 0)
    new_v, new_i = [], []
    for t in range(_TOPK):
        mv = jnp.min(wv, axis=0, keepdims=True)
        r = jnp.min(jnp.where(wv == mv, rows8, _BIGI), axis=0, keepdims=True)
        sel = rows8 == r
        gi = jnp.min(jnp.where(sel, wi, _BIGI), axis=0, keepdims=True)
        new_v.append(mv)
        new_i.append(gi)
        wv = jnp.where(sel, jnp.inf, wv)
    vals_s[0:4, :] = jnp.concatenate(new_v, axis=0)
    idx_s[0:4, :] = jnp.concatenate(new_i, axis=0)

    @pl.when(k == nk)
    def _finish():
        v4 = vals_s[0:4, :]
        dist = jnp.sqrt(jnp.maximum(v4, 1e-12))
        e = jnp.exp(dist[0:1, :] - dist)
        w = e / jnp.sum(e, axis=0, keepdims=True)
        w_ref[0:4, :] = w
        w_ref[4:8, :] = jnp.zeros((4, w.shape[1]), jnp.float32)
        idx_ref[0:4, :] = idx_s[0:4, :]
        idx_ref[4:8, :] = jnp.zeros((4, w.shape[1]), jnp.int32)


def _dist_topk(q, m):
    Q, D = q.shape
    n_valid = m.shape[0]
    nk = (n_valid + _BK - 1) // _BK
    body = functools.partial(_dist_topk_body, nk=nk, bk=_BK, n_valid=n_valid)
    return pl.pallas_call(
        body,
        grid=(nk + 1,),
        in_specs=[
            pl.BlockSpec((Q, D), lambda k: (0, 0)),
            pl.BlockSpec((_BK, D), lambda k: (jnp.minimum(k, nk - 1), 0)),
        ],
        out_specs=[
            pl.BlockSpec((8, Q), lambda k: (0, 0)),
            pl.BlockSpec((8, Q), lambda k: (0, 0)),
        ],
        out_shape=[
            jax.ShapeDtypeStruct((8, Q), jnp.float32),
            jax.ShapeDtypeStruct((8, Q), jnp.int32),
        ],
        scratch_shapes=[
            pltpu.VMEM((8, Q), jnp.float32),
            pltpu.VMEM((8, Q), jnp.int32),
            pltpu.VMEM((8, Q), jnp.float32),
            pltpu.VMEM((_BK, Q), jnp.float32),
            pltpu.VMEM((_BK, Q), jnp.float32),
        ],
        compiler_params=pltpu.CompilerParams(
            dimension_semantics=("arbitrary",)),
    )(q, m)


def _sc_gather(table, idx_flat):
    B = idx_flat.shape[0]
    D = table.shape[1]
    info = plsc.get_sparse_core_info()
    nw = info.num_cores * info.num_subcores
    b_per_w = B // nw
    ch = 32
    nch = b_per_w // ch
    mesh = plsc.VectorSubcoreMesh(core_axis_name="c", subcore_axis_name="s")

    @functools.partial(
        pl.kernel, mesh=mesh,
        out_type=jax.ShapeDtypeStruct((B, D), jnp.float32),
        scratch_types=[
            pltpu.VMEM((ch,), jnp.int32),
            pltpu.VMEM((ch, D), jnp.float32),
            pltpu.SemaphoreType.DMA,
        ],
    )
    def k(table_hbm, idx_hbm, out_hbm, idx_v, rows_v, sem):
        wid = lax.axis_index("s") * info.num_cores + lax.axis_index("c")
        for c in range(nch):
            base = wid * b_per_w + c * ch
            pltpu.sync_copy(idx_hbm.at[pl.ds(base, ch)], idx_v)
            pltpu.async_copy(table_hbm.at[idx_v], rows_v, sem).wait()
            pltpu.sync_copy(rows_v, out_hbm.at[pl.ds(base, ch)])

    return k(table, idx_flat)


def _wsum_body(g_ref, w_ref, o_ref):
    w = w_ref[...]
    acc = w[:, 0:1] * g_ref[0]
    for j in range(1, _TOPK):
        acc = acc + w[:, j:j + 1] * g_ref[j]
    o_ref[...] = acc


def _wsum(g4, w_q):
    _, Q, D = g4.shape
    BQ = 256
    return pl.pallas_call(
        _wsum_body,
        grid=(Q // BQ,),
        in_specs=[
            pl.BlockSpec((_TOPK, BQ, D), lambda i: (0, i, 0)),
            pl.BlockSpec((BQ, 8), lambda i: (i, 0)),
        ],
        out_specs=pl.BlockSpec((BQ, D), lambda i: (i, 0)),
        out_shape=jax.ShapeDtypeStruct((Q, D), jnp.float32),
    )(g4, w_q)


def kernel(query_seq, matching_set, synth_set, topk):
    Q, D = query_seq.shape
    w8, idx8 = _dist_topk(query_seq, matching_set)
    idx_flat = idx8[0:_TOPK, :].reshape(_TOPK * Q)
    g = _sc_gather(synth_set, idx_flat)
    g4 = g.reshape(_TOPK, Q, D)
    out = _wsum(g4, w8.T)
    return out


# speedup vs baseline: 1.2170x; 1.0095x over previous
import functools

import jax
import jax.numpy as jnp
from jax import lax
from jax.experimental import pallas as pl
from jax.experimental.pallas import tpu as pltpu
from jax.experimental.pallas import tpu_sc as plsc

_BK = 1792
_TOPK = 4
_BIGI = 2**30


def _dist_topk_body(q_ref, m_ref, w_ref, idx_ref, vals_s, idx_s, qsq_s,
                    d2a_s, d2b_s, *, nk, bk, n_valid):
    k = pl.program_id(0)
    par = lax.rem(k, 2)

    @pl.when(k == 0)
    def _init():
        vals_s[...] = jnp.full(vals_s.shape, jnp.inf, jnp.float32)
        idx_s[...] = jnp.zeros(idx_s.shape, jnp.int32)
        q0 = q_ref[...]
        ones = jnp.ones((8, q0.shape[1]), jnp.float32)
        qsq_s[...] = lax.dot_general(ones, q0 * q0, (((1,), (1,)), ((), ())),
                                     preferred_element_type=jnp.float32)
        d2b_s[...] = jnp.full(d2b_s.shape, jnp.inf, jnp.float32)

    q = q_ref[...]
    Q = q.shape[0]
    nch = 2
    ch = bk // nch

    def _phase(wr_ref, rd_ref):
        def emit_mm_chunk(c):
            mc = m_ref[ch * c:ch * (c + 1), :]
            msq_c = jnp.sum(mc * mc, axis=1, keepdims=True)
            mm_c = lax.dot_general(mc, q, (((1,), (1,)), ((), ())),
                                   preferred_element_type=jnp.float32,
                                   precision=lax.Precision.DEFAULT)
            d2_c = (qsq_s[0:1, :] + msq_c) - 2.0 * mm_c
            rows_c = lax.broadcasted_iota(jnp.int32, d2_c.shape, 0)
            d2_c = jnp.where(rows_c + (k * bk + c * ch) < n_valid,
                             d2_c, jnp.inf)
            wr_ref[ch * c:ch * (c + 1), :] = d2_c

        work = rd_ref[...]
        rows = lax.broadcasted_iota(jnp.int32, work.shape, 0)
        base_prev = (k - 1) * bk
        emit_mm_chunk(0)
        for t in range(_TOPK):
            mv = jnp.min(work, axis=0, keepdims=True)
            r = jnp.min(jnp.where(work == mv, rows, _BIGI),
                        axis=0, keepdims=True)
            vals_s[4 + t:5 + t, :] = mv
            idx_s[4 + t:5 + t, :] = r + base_prev
            if t < _TOPK - 1:
                work = jnp.where(rows == r, jnp.inf, work)
                if t + 1 < nch:
                    emit_mm_chunk(t + 1)

    @pl.when(par == 0)
    def _even():
        _phase(d2a_s, d2b_s)

    @pl.when(par == 1)
    def _odd():
        _phase(d2b_s, d2a_s)

    wv = vals_s[...]
    wi = idx_s[...]
    rows8 = lax.broadcasted_iota(jnp.int32, wv.shape, 0)
    new_v, new_i = [], []
    for t in range(_TOPK):
        mv = jnp.min(wv, axis=0, keepdims=True)
        r = jnp.min(jnp.where(wv == mv, rows8, _BIGI), axis=0, keepdims=True)
        sel = rows8 == r
        gi = jnp.min(jnp.where(sel, wi, _BIGI), axis=0, keepdims=True)
        new_v.append(mv)
        new_i.append(gi)
        wv = jnp.where(sel, jnp.inf, wv)
    vals_s[0:4, :] = jnp.concatenate(new_v, axis=0)
    idx_s[0:4, :] = jnp.concatenate(new_i, axis=0)

    @pl.when(k == nk)
    def _finish():
        v4 = vals_s[0:4, :]
        dist = jnp.sqrt(jnp.maximum(v4, 1e-12))
        e = jnp.exp(dist[0:1, :] - dist)
        w = e / jnp.sum(e, axis=0, keepdims=True)
        w_ref[0:4, :] = w
        w_ref[4:8, :] = jnp.zeros((4, w.shape[1]), jnp.float32)
        idx_ref[0:4, :] = idx_s[0:4, :]
        idx_ref[4:8, :] = jnp.zeros((4, w.shape[1]), jnp.int32)


def _dist_topk(q, m):
    Q, D = q.shape
    n_valid = m.shape[0]
    nk = (n_valid + _BK - 1) // _BK
    body = functools.partial(_dist_topk_body, nk=nk, bk=_BK, n_valid=n_valid)
    return pl.pallas_call(
        body,
        grid=(nk + 1,),
        in_specs=[
            pl.BlockSpec((Q, D), lambda k: (0, 0)),
            pl.BlockSpec((_BK, D), lambda k: (jnp.minimum(k, nk - 1), 0)),
        ],
        out_specs=[
            pl.BlockSpec((8, Q), lambda k: (0, 0)),
            pl.BlockSpec((8, Q), lambda k: (0, 0)),
        ],
        out_shape=[
            jax.ShapeDtypeStruct((8, Q), jnp.float32),
            jax.ShapeDtypeStruct((8, Q), jnp.int32),
        ],
        scratch_shapes=[
            pltpu.VMEM((8, Q), jnp.float32),
            pltpu.VMEM((8, Q), jnp.int32),
            pltpu.VMEM((8, Q), jnp.float32),
            pltpu.VMEM((_BK, Q), jnp.float32),
            pltpu.VMEM((_BK, Q), jnp.float32),
        ],
        compiler_params=pltpu.CompilerParams(
            dimension_semantics=("arbitrary",)),
    )(q, m)


def _sc_gather(table, idx_flat):
    B = idx_flat.shape[0]
    D = table.shape[1]
    info = plsc.get_sparse_core_info()
    nw = info.num_cores * info.num_subcores
    b_per_w = B // nw
    ch = 32
    nch = b_per_w // ch
    mesh = plsc.VectorSubcoreMesh(core_axis_name="c", subcore_axis_name="s")

    @functools.partial(
        pl.kernel, mesh=mesh,
        out_type=jax.ShapeDtypeStruct((B, D), jnp.float32),
        scratch_types=[
            pltpu.VMEM((ch,), jnp.int32),
            pltpu.VMEM((ch, D), jnp.float32),
            pltpu.SemaphoreType.DMA,
        ],
    )
    def k(table_hbm, idx_hbm, out_hbm, idx_v, rows_v, sem):
        wid = lax.axis_index("s") * info.num_cores + lax.axis_index("c")
        for c in range(nch):
            base = wid * b_per_w + c * ch
            pltpu.sync_copy(idx_hbm.at[pl.ds(base, ch)], idx_v)
            pltpu.async_copy(table_hbm.at[idx_v], rows_v, sem).wait()
            pltpu.sync_copy(rows_v, out_hbm.at[pl.ds(base, ch)])

    return k(table, idx_flat)


def _wsum_body(g_ref, w_ref, o_ref):
    w = w_ref[...]
    acc = w[:, 0:1] * g_ref[0]
    for j in range(1, _TOPK):
        acc = acc + w[:, j:j + 1] * g_ref[j]
    o_ref[...] = acc


def _wsum(g4, w_q):
    _, Q, D = g4.shape
    BQ = 256
    return pl.pallas_call(
        _wsum_body,
        grid=(Q // BQ,),
        in_specs=[
            pl.BlockSpec((_TOPK, BQ, D), lambda i: (0, i, 0)),
            pl.BlockSpec((BQ, 8), lambda i: (i, 0)),
        ],
        out_specs=pl.BlockSpec((BQ, D), lambda i: (i, 0)),
        out_shape=jax.ShapeDtypeStruct((Q, D), jnp.float32),
    )(g4, w_q)


def kernel(query_seq, matching_set, synth_set, topk):
    Q, D = query_seq.shape
    w8, idx8 = _dist_topk(query_seq, matching_set)
    idx_flat = idx8[0:_TOPK, :].reshape(_TOPK * Q)
    g = _sc_gather(synth_set, idx_flat)
    g4 = g.reshape(_TOPK, Q, D)
    out = _wsum(g4, w8.T)
    return out


# speedup vs baseline: 1.2284x; 1.0094x over previous
import functools

import jax
import jax.numpy as jnp
from jax import lax
from jax.experimental import pallas as pl
from jax.experimental.pallas import tpu as pltpu
from jax.experimental.pallas import tpu_sc as plsc

_BK = 1792
_TOPK = 4
_BIGI = 2**30


def _dist_topk_body(q_ref, m_ref, w_ref, idx_ref, vals_s, idx_s, qsq_s,
                    d2a_s, d2b_s, *, nk, bk, n_valid):
    k = pl.program_id(0)
    par = lax.rem(k, 2)

    @pl.when(k == 0)
    def _init():
        vals_s[...] = jnp.full(vals_s.shape, jnp.inf, jnp.float32)
        idx_s[...] = jnp.zeros(idx_s.shape, jnp.int32)
        q0 = q_ref[...]
        ones = jnp.ones((8, q0.shape[1]), jnp.float32)
        qsq_s[...] = lax.dot_general(ones, q0 * q0, (((1,), (1,)), ((), ())),
                                     preferred_element_type=jnp.float32)
        d2b_s[...] = jnp.full(d2b_s.shape, jnp.inf, jnp.float32)

    q = q_ref[...]
    Q = q.shape[0]
    nch = 1
    ch = bk // nch

    def _phase(wr_ref, rd_ref):
        def emit_mm_chunk(c):
            mc = m_ref[ch * c:ch * (c + 1), :]
            msq_c = jnp.sum(mc * mc, axis=1, keepdims=True)
            mm_c = lax.dot_general(mc, q, (((1,), (1,)), ((), ())),
                                   preferred_element_type=jnp.float32,
                                   precision=lax.Precision.DEFAULT)
            d2_c = (qsq_s[0:1, :] + msq_c) - 2.0 * mm_c
            rows_c = lax.broadcasted_iota(jnp.int32, d2_c.shape, 0)
            d2_c = jnp.where(rows_c + (k * bk + c * ch) < n_valid,
                             d2_c, jnp.inf)
            wr_ref[ch * c:ch * (c + 1), :] = d2_c

        work = rd_ref[...]
        rows = lax.broadcasted_iota(jnp.int32, work.shape, 0)
        base_prev = (k - 1) * bk
        emit_mm_chunk(0)
        for t in range(_TOPK):
            mv = jnp.min(work, axis=0, keepdims=True)
            r = jnp.min(jnp.where(work == mv, rows, _BIGI),
                        axis=0, keepdims=True)
            vals_s[4 + t:5 + t, :] = mv
            idx_s[4 + t:5 + t, :] = r + base_prev
            if t < _TOPK - 1:
                work = jnp.where(rows == r, jnp.inf, work)
                if t + 1 < nch:
                    emit_mm_chunk(t + 1)

    @pl.when(par == 0)
    def _even():
        _phase(d2a_s, d2b_s)

    @pl.when(par == 1)
    def _odd():
        _phase(d2b_s, d2a_s)

    wv = vals_s[...]
    wi = idx_s[...]
    rows8 = lax.broadcasted_iota(jnp.int32, wv.shape, 0)
    new_v, new_i = [], []
    for t in range(_TOPK):
        mv = jnp.min(wv, axis=0, keepdims=True)
        r = jnp.min(jnp.where(wv == mv, rows8, _BIGI), axis=0, keepdims=True)
        sel = rows8 == r
        gi = jnp.min(jnp.where(sel, wi, _BIGI), axis=0, keepdims=True)
        new_v.append(mv)
        new_i.append(gi)
        wv = jnp.where(sel, jnp.inf, wv)
    vals_s[0:4, :] = jnp.concatenate(new_v, axis=0)
    idx_s[0:4, :] = jnp.concatenate(new_i, axis=0)

    @pl.when(k == nk)
    def _finish():
        v4 = vals_s[0:4, :]
        dist = jnp.sqrt(jnp.maximum(v4, 1e-12))
        e = jnp.exp(dist[0:1, :] - dist)
        w = e / jnp.sum(e, axis=0, keepdims=True)
        w_ref[0:4, :] = w
        w_ref[4:8, :] = jnp.zeros((4, w.shape[1]), jnp.float32)
        idx_ref[0:4, :] = idx_s[0:4, :]
        idx_ref[4:8, :] = jnp.zeros((4, w.shape[1]), jnp.int32)


def _dist_topk(q, m):
    Q, D = q.shape
    n_valid = m.shape[0]
    nk = (n_valid + _BK - 1) // _BK
    body = functools.partial(_dist_topk_body, nk=nk, bk=_BK, n_valid=n_valid)
    return pl.pallas_call(
        body,
        grid=(nk + 1,),
        in_specs=[
            pl.BlockSpec((Q, D), lambda k: (0, 0)),
            pl.BlockSpec((_BK, D), lambda k: (jnp.minimum(k, nk - 1), 0)),
        ],
        out_specs=[
            pl.BlockSpec((8, Q), lambda k: (0, 0)),
            pl.BlockSpec((8, Q), lambda k: (0, 0)),
        ],
        out_shape=[
            jax.ShapeDtypeStruct((8, Q), jnp.float32),
            jax.ShapeDtypeStruct((8, Q), jnp.int32),
        ],
        scratch_shapes=[
            pltpu.VMEM((8, Q), jnp.float32),
            pltpu.VMEM((8, Q), jnp.int32),
            pltpu.VMEM((8, Q), jnp.float32),
            pltpu.VMEM((_BK, Q), jnp.float32),
            pltpu.VMEM((_BK, Q), jnp.float32),
        ],
        compiler_params=pltpu.CompilerParams(
            dimension_semantics=("arbitrary",)),
    )(q, m)


def _sc_gather(table, idx_flat):
    B = idx_flat.shape[0]
    D = table.shape[1]
    info = plsc.get_sparse_core_info()
    nw = info.num_cores * info.num_subcores
    b_per_w = B // nw
    ch = 32
    nch = b_per_w // ch
    mesh = plsc.VectorSubcoreMesh(core_axis_name="c", subcore_axis_name="s")

    @functools.partial(
        pl.kernel, mesh=mesh,
        out_type=jax.ShapeDtypeStruct((B, D), jnp.float32),
        scratch_types=[
            pltpu.VMEM((ch,), jnp.int32),
            pltpu.VMEM((ch, D), jnp.float32),
            pltpu.SemaphoreType.DMA,
        ],
    )
    def k(table_hbm, idx_hbm, out_hbm, idx_v, rows_v, sem):
        wid = lax.axis_index("s") * info.num_cores + lax.axis_index("c")
        for c in range(nch):
            base = wid * b_per_w + c * ch
            pltpu.sync_copy(idx_hbm.at[pl.ds(base, ch)], idx_v)
            pltpu.async_copy(table_hbm.at[idx_v], rows_v, sem).wait()
            pltpu.sync_copy(rows_v, out_hbm.at[pl.ds(base, ch)])

    return k(table, idx_flat)


def _wsum_body(g_ref, w_ref, o_ref):
    w = w_ref[...]
    acc = w[:, 0:1] * g_ref[0]
    for j in range(1, _TOPK):
        acc = acc + w[:, j:j + 1] * g_ref[j]
    o_ref[...] = acc


def _wsum(g4, w_q):
    _, Q, D = g4.shape
    BQ = 256
    return pl.pallas_call(
        _wsum_body,
        grid=(Q // BQ,),
        in_specs=[
            pl.BlockSpec((_TOPK, BQ, D), lambda i: (0, i, 0)),
            pl.BlockSpec((BQ, 8), lambda i: (i, 0)),
        ],
        out_specs=pl.BlockSpec((BQ, D), lambda i: (i, 0)),
        out_shape=jax.ShapeDtypeStruct((Q, D), jnp.float32),
    )(g4, w_q)


def kernel(query_seq, matching_set, synth_set, topk):
    Q, D = query_seq.shape
    w8, idx8 = _dist_topk(query_seq, matching_set)
    idx_flat = idx8[0:_TOPK, :].reshape(_TOPK * Q)
    g = _sc_gather(synth_set, idx_flat)
    g4 = g.reshape(_TOPK, Q, D)
    out = _wsum(g4, w8.T)
    return out
